# baseline (speedup 1.0000x reference)
import functools
import jax
import jax.numpy as jnp
from jax import lax
from jax.experimental import pallas as pl
from jax.experimental.pallas import tpu as pltpu
from jax.experimental.pallas import tpu_sc as plsc

S = 128
B = 2
M = 4
EPS = 1e-4

NC, NS, L = 2, 16, 16
NPTS_PAD = 100352
ROWS_PT = NPTS_PAD // (NC * NS * 128) * NC
QROWS = 49
QUART = S * S * S // 4
TSLICE = QUART // NS


def _sc_scatter_body(idx_hbm, feat_hbm, sum_hbm, cnt_hbm,
                     idxb, featb, relb, onesb, zbuf, spm_sum, spm_cnt):
    cid = lax.axis_index("c")
    tid = lax.axis_index("s")
    pltpu.sync_copy(idx_hbm.at[tid], idxb)
    pltpu.sync_copy(feat_hbm.at[tid], featb)
    for i in range(128 // L):
        onesb[pl.ds(i * L, L)] = jnp.ones((L,), jnp.float32)
    def _z(i, carry):
        zbuf[pl.ds(i * L, L)] = jnp.zeros((L,), jnp.float32)
        return carry
    lax.fori_loop(0, TSLICE // L, _z, 0)

    for q in range(4):
        base = cid * (S * S * S) + q * QUART
        pltpu.sync_copy(zbuf, spm_sum.at[pl.ds(tid * TSLICE, TSLICE)])
        pltpu.sync_copy(zbuf, spm_cnt.at[pl.ds(tid * TSLICE, TSLICE)])
        plsc.subcore_barrier()

        def _rel(j, carry):
            for i in range(128 // L):
                v = idxb[j, pl.ds(i * L, L)] - base
                ok = (v >= 0) & (v < QUART)
                relb[j, pl.ds(i * L, L)] = jnp.where(ok, v, QUART)
            return carry
        lax.fori_loop(0, QROWS, _rel, 0, unroll=2)

        for j in range(QROWS):
            pltpu.sync_copy(featb.at[j], spm_sum.at[relb.at[j]], add=True)
            pltpu.sync_copy(onesb, spm_cnt.at[relb.at[j]], add=True)
        plsc.subcore_barrier()

        pltpu.sync_copy(spm_sum.at[pl.ds(tid * TSLICE, TSLICE)],
                        sum_hbm.at[pl.ds(base + tid * TSLICE, TSLICE)])
        pltpu.sync_copy(spm_cnt.at[pl.ds(tid * TSLICE, TSLICE)],
                        cnt_hbm.at[pl.ds(base + tid * TSLICE, TSLICE)])
        plsc.subcore_barrier()


def _sc_voxelize(lin_idx, feat):
    idx2d = lin_idx.reshape(NS, QROWS, 128)
    feat2d = feat.reshape(NS, QROWS, 128)
    mesh = plsc.VectorSubcoreMesh(core_axis_name="c", subcore_axis_name="s",
                                  num_cores=NC, num_subcores=NS)
    f = pl.kernel(
        _sc_scatter_body,
        mesh=mesh,
        out_type=[jax.ShapeDtypeStruct((B * S * S * S,), jnp.float32),
                  jax.ShapeDtypeStruct((B * S * S * S,), jnp.float32)],
        scratch_types=[
            pltpu.VMEM((QROWS, 128), jnp.int32),
            pltpu.VMEM((QROWS, 128), jnp.float32),
            pltpu.VMEM((QROWS, 128), jnp.int32),
            pltpu.VMEM((128,), jnp.float32),
            pltpu.VMEM((TSLICE,), jnp.float32),
            pltpu.VMEM_SHARED((QUART + 128,), jnp.float32),
            pltpu.VMEM_SHARED((QUART + 128,), jnp.float32),
        ],
    )
    return f(idx2d, feat2d)


def _sh(x, t, axis):
    if t == 0:
        return x
    pad = [(0, 0)] * x.ndim
    pad[axis] = (0, t)
    xp = jnp.pad(x, pad)
    idx = [slice(None)] * x.ndim
    idx[axis] = slice(t, t + x.shape[axis])
    return xp[tuple(idx)]


def _tail_kernel(y_ref, m_ref, bn1w_ref, bn1b_ref, cw1_ref, bn2w_ref,
                 bn2b_ref, cw2_ref, l8_ref, r8_ref, l8m_ref, r8m_ref,
                 l2_ref, r2_ref, out_ref):
    n1 = jnp.maximum(m_ref[0].sum() + m_ref[1].sum(), 1.0)
    y = [[y_ref[b * M + c] for c in range(M)] for b in range(B)]
    z = [[None] * M for _ in range(B)]
    for c in range(M):
        s1 = sum(y[b][c].sum() for b in range(B))
        s2 = sum((y[b][c] * y[b][c]).sum() for b in range(B))
        mean = s1 / n1
        var = s2 / n1 - mean * mean
        a = bn1w_ref[c] * lax.rsqrt(var + EPS)
        cc = bn1b_ref[c] - mean * a
        for b in range(B):
            z[b][c] = jnp.maximum((y[b][c] * a + cc) * m_ref[b], 0.0)

    y8 = [[None] * M for _ in range(B)]
    m8 = [None] * B
    for b in range(B):
        mm = jnp.maximum(m_ref[b], _sh(m_ref[b], 1, 2))
        mm = jnp.maximum(mm, _sh(mm, 1, 1))
        mm = jnp.maximum(mm, _sh(mm, 1, 0))
        m8[b] = jnp.stack([
            jnp.dot(jnp.dot(l8m_ref[...], mm[4 * d]), r8m_ref[...],
                    preferred_element_type=jnp.float32) for d in range(8)])
        for co in range(M):
            acc = jnp.zeros((32, 32, 32), jnp.float32)
            for td in range(2):
                for th in range(2):
                    for tw in range(2):
                        for ci in range(M):
                            w = cw1_ref[td * 4 + th * 2 + tw, ci * M + co]
                            zs = _sh(_sh(_sh(z[b][ci], tw, 2), th, 1), td, 0)
                            acc = acc + zs * w
            planes = []
            for d in range(8):
                p = acc[4 * d] + acc[4 * d + 2]
                planes.append(jnp.dot(jnp.dot(l8_ref[...], p), r8_ref[...],
                                      preferred_element_type=jnp.float32))
            y8[b][co] = jnp.stack(planes) * 0.5

    n2 = jnp.maximum(sum(jnp.sum(m8[b]) for b in range(B)), 1.0)
    z2 = [[None] * M for _ in range(B)]
    for c in range(M):
        s1 = sum(y8[b][c].sum() for b in range(B))
        s2 = sum((y8[b][c] * y8[b][c]).sum() for b in range(B))
        mean = s1 / n2
        var = s2 / n2 - mean * mean
        a = bn2w_ref[c] * lax.rsqrt(var + EPS)
        cc = bn2b_ref[c] - mean * a
        for b in range(B):
            z2[b][c] = jnp.maximum((y8[b][c] * a + cc) * m8[b], 0.0)

    for b in range(B):
        for co in range(M):
            acc = jnp.zeros((8, 8, 8), jnp.float32)
            for td in range(2):
                for th in range(2):
                    for tw in range(2):
                        for ci in range(M):
                            w = cw2_ref[td * 4 + th * 2 + tw, ci * M + co]
                            zs = _sh(_sh(_sh(z2[b][ci], tw, 2), th, 1), td, 0)
                            acc = acc + zs * w
            planes = []
            for d in range(2):
                p = acc[4 * d] + acc[4 * d + 2]
                planes.append(jnp.dot(jnp.dot(l2_ref[...], p), r2_ref[...],
                                      preferred_element_type=jnp.float32))
            out_ref[b * M + co] = jnp.stack(planes) * 0.5


def _tail(y32, m32, bn_w1, bn_b1, W_conv1, bn_w2, bn_b2, W_conv2):
    idx8 = jnp.arange(32)
    l8 = ((idx8[None, :] == 4 * jnp.arange(8)[:, None]) |
          (idx8[None, :] == 4 * jnp.arange(8)[:, None] + 2)).astype(jnp.float32) * 0.5
    r8 = l8.T
    l8m = (idx8[None, :] == 4 * jnp.arange(8)[:, None]).astype(jnp.float32)
    r8m = l8m.T
    idx2 = jnp.arange(8)
    l2 = ((idx2[None, :] == 4 * jnp.arange(2)[:, None]) |
          (idx2[None, :] == 4 * jnp.arange(2)[:, None] + 2)).astype(jnp.float32) * 0.5
    r2 = l2.T
    cw1 = W_conv1.reshape(8, 16)
    cw2 = W_conv2.reshape(8, 16)
    smem = pl.BlockSpec(memory_space=pltpu.SMEM)
    out = pl.pallas_call(
        _tail_kernel,
        out_shape=jax.ShapeDtypeStruct((B * M, 2, 2, 2), jnp.float32),
        in_specs=[pl.BlockSpec(memory_space=pltpu.VMEM)] * 2 +
                 [smem, smem, smem, smem, smem, smem] +
                 [pl.BlockSpec(memory_space=pltpu.VMEM)] * 6,
        out_specs=pl.BlockSpec(memory_space=pltpu.VMEM),
    )(y32.reshape(B * M, 32, 32, 32), m32, bn_w1, bn_b1, cw1,
      bn_w2, bn_b2, cw2, l8, r8, l8m, r8m, l2, r2)
    return out.reshape(B, M * 8)


def kernel(point_cloud, W_sub, bn_w0, bn_b0, W_conv0, bn_w1, bn_b1, W_conv1,
           bn_w2, bn_b2, W_conv2):
    pc = point_cloud
    ix = pc[:, 0].astype(jnp.int32)
    iy = pc[:, 1].astype(jnp.int32)
    iz = pc[:, 2].astype(jnp.int32)
    ib = pc[:, 3].astype(jnp.int32)
    lin = ((ib * S + ix) * S + iy) * S + iz
    npts = lin.shape[0]
    lin_pad = jnp.full((NPTS_PAD,), B * S * S * S, jnp.int32).at[:npts].set(lin)
    feat_pad = jnp.zeros((NPTS_PAD,), jnp.float32).at[:npts].set(pc[:, 4])
    dense, cnt = _sc_voxelize(lin_pad, feat_pad)
    dense = dense.reshape(B, 1, S, S, S)
    mask = (cnt > 0).astype(jnp.float32).reshape(B, 1, S, S, S)

    x = lax.conv_general_dilated(
        dense, W_sub, (1, 1, 1), 'SAME',
        dimension_numbers=('NCDHW', 'DHWIO', 'NCDHW')) * mask
    n0 = jnp.maximum(jnp.sum(mask), 1.0)
    s1 = jnp.sum(x, axis=(0, 2, 3, 4))
    s2 = jnp.sum(x * x, axis=(0, 2, 3, 4))
    mean = s1 / n0
    var = s2 / n0 - mean * mean
    a = bn_w0 / jnp.sqrt(var + EPS)
    c = bn_b0 - mean * a
    z = jnp.maximum((x * a[None, :, None, None, None]
                     + c[None, :, None, None, None]) * mask, 0.0)
    y64 = lax.conv_general_dilated(
        z, W_conv0, (2, 2, 2), 'VALID',
        dimension_numbers=('NCDHW', 'DHWIO', 'NCDHW'))
    y32 = lax.reduce_window(y64, 0.0, lax.add, (1, 1, 2, 2, 2),
                            (1, 1, 2, 2, 2), 'VALID') / 8.0
    m32 = lax.reduce_window(mask, -jnp.inf, lax.max, (1, 1, 4, 4, 4),
                            (1, 1, 4, 4, 4), 'VALID')
    return _tail(y32, m32[:, 0], bn_w1, bn_b1, W_conv1, bn_w2, bn_b2, W_conv2)


# speedup vs baseline: 6.3103x; 6.3103x over previous
import functools
import jax
import jax.numpy as jnp
from jax import lax
from jax.experimental import pallas as pl
from jax.experimental.pallas import tpu as pltpu
from jax.experimental.pallas import tpu_sc as plsc

S = 128
B = 2
M = 4
EPS = 1e-4

NC, NS, L = 2, 16, 16
NPTS_PAD = 100352
ROWS_PT = NPTS_PAD // (NC * NS * 128) * NC
QROWS = 49
QUART = S * S * S // 4
TSLICE = QUART // NS


def _sc_scatter_body(idx_hbm, feat_hbm, sum_hbm, cnt_hbm,
                     idxb, featb, relb, onesb, zbuf, spm_sum, spm_cnt):
    cid = lax.axis_index("c")
    tid = lax.axis_index("s")
    pltpu.sync_copy(idx_hbm.at[tid], idxb)
    pltpu.sync_copy(feat_hbm.at[tid], featb)
    for i in range(128 // L):
        onesb[pl.ds(i * L, L)] = jnp.ones((L,), jnp.float32)
    def _z(i, carry):
        zbuf[pl.ds(i * L, L)] = jnp.zeros((L,), jnp.float32)
        return carry
    lax.fori_loop(0, TSLICE // L, _z, 0)

    for q in range(4):
        base = cid * (S * S * S) + q * QUART
        pltpu.sync_copy(zbuf, spm_sum.at[pl.ds(tid * TSLICE, TSLICE)])
        pltpu.sync_copy(zbuf, spm_cnt.at[pl.ds(tid * TSLICE, TSLICE)])
        plsc.subcore_barrier()

        def _rel(j, carry):
            for i in range(128 // L):
                v = idxb[j, pl.ds(i * L, L)] - base
                ok = (v >= 0) & (v < QUART)
                relb[j, pl.ds(i * L, L)] = jnp.where(ok, v, QUART)
            return carry
        lax.fori_loop(0, QROWS, _rel, 0, unroll=2)

        for j in range(QROWS):
            pltpu.sync_copy(featb.at[j], spm_sum.at[relb.at[j]], add=True)
            pltpu.sync_copy(onesb, spm_cnt.at[relb.at[j]], add=True)
        plsc.subcore_barrier()

        pltpu.sync_copy(spm_sum.at[pl.ds(tid * TSLICE, TSLICE)],
                        sum_hbm.at[pl.ds(base + tid * TSLICE, TSLICE)])
        pltpu.sync_copy(spm_cnt.at[pl.ds(tid * TSLICE, TSLICE)],
                        cnt_hbm.at[pl.ds(base + tid * TSLICE, TSLICE)])
        plsc.subcore_barrier()


def _sc_voxelize(lin_idx, feat):
    idx2d = lin_idx.reshape(NS, QROWS, 128)
    feat2d = feat.reshape(NS, QROWS, 128)
    mesh = plsc.VectorSubcoreMesh(core_axis_name="c", subcore_axis_name="s",
                                  num_cores=NC, num_subcores=NS)
    f = pl.kernel(
        _sc_scatter_body,
        mesh=mesh,
        out_type=[jax.ShapeDtypeStruct((B * S * S * S,), jnp.float32),
                  jax.ShapeDtypeStruct((B * S * S * S,), jnp.float32)],
        scratch_types=[
            pltpu.VMEM((QROWS, 128), jnp.int32),
            pltpu.VMEM((QROWS, 128), jnp.float32),
            pltpu.VMEM((QROWS, 128), jnp.int32),
            pltpu.VMEM((128,), jnp.float32),
            pltpu.VMEM((TSLICE,), jnp.float32),
            pltpu.VMEM_SHARED((QUART + 128,), jnp.float32),
            pltpu.VMEM_SHARED((QUART + 128,), jnp.float32),
        ],
    )
    return f(idx2d, feat2d)


DBLK = 16


def _make_volp(dm1_ref, d0_ref, dp1_ref, k, nk):
    zero = jnp.zeros((1, S, S), jnp.float32)
    prev = jnp.where(k > 0, dm1_ref[0, DBLK - 1:DBLK], zero)
    nxt = jnp.where(k < nk - 1, dp1_ref[0, 0:1], zero)
    vol = jnp.concatenate([prev, d0_ref[0], nxt], axis=0)
    return jnp.pad(vol, ((0, 0), (1, 1), (1, 1)))


def _conv_plane(volp, d, w_ref):
    taps = [volp[d + dd, dh:dh + S, dw:dw + S]
            for dd in range(3) for dh in range(3) for dw in range(3)]
    out = []
    for c in range(M):
        acc = taps[0] * w_ref[0, c]
        for t in range(1, 27):
            acc = acc + taps[t] * w_ref[t, c]
        out.append(acc)
    return out


def _stats_update(out_ref, first, s1, s2, n):
    row = lax.broadcasted_iota(jnp.int32, (8, 128), 0)
    lane = lax.broadcasted_iota(jnp.int32, (8, 128), 1)
    upd = jnp.where((row == 2) & (lane == 0), n, 0.0)
    for c in range(M):
        upd = jnp.where((row == 0) & (lane == c), s1[c], upd)
        upd = jnp.where((row == 1) & (lane == c), s2[c], upd)
    @pl.when(first)
    def _():
        out_ref[...] = jnp.zeros((8, 128), jnp.float32)
    out_ref[...] += upd


def _k1_kernel(dm1, d0, dp1, ck, w_ref, out_ref):
    b, k = pl.program_id(0), pl.program_id(1)
    volp = _make_volp(dm1, d0, dp1, k, S // DBLK)
    s1 = [jnp.float32(0.0)] * M
    s2 = [jnp.float32(0.0)] * M
    n = jnp.float32(0.0)
    for d in range(DBLK):
        x = _conv_plane(volp, d, w_ref)
        m = (ck[0, d] > 0).astype(jnp.float32)
        n = n + m.sum()
        for c in range(M):
            xm = x[c] * m
            s1[c] = s1[c] + xm.sum()
            s2[c] = s2[c] + (xm * xm).sum()
    _stats_update(out_ref, (b == 0) & (k == 0), s1, s2, n)


def _k2_kernel(dm1, d0, dp1, ck, stats_ref, w_ref, bw_ref, bb_ref, cw_ref,
               l0_ref, r0_ref, y_ref, m_ref):
    b, k = pl.program_id(0), pl.program_id(1)
    volp = _make_volp(dm1, d0, dp1, k, S // DBLK)
    n = jnp.maximum(stats_ref[2, 0], 1.0)
    aff = []
    for c in range(M):
        mean = stats_ref[0, c] / n
        var = stats_ref[1, c] / n - mean * mean
        a = bw_ref[c] * lax.rsqrt(var + EPS)
        aff.append((a, bb_ref[c] - mean * a))
    for f in range(DBLK // 4):
        d0i = 4 * f
        z = [[None] * 4 for _ in range(M)]
        for p in range(4):
            x = _conv_plane(volp, d0i + p, w_ref)
            m = (ck[0, d0i + p] > 0).astype(jnp.float32)
            for c in range(M):
                a, cc = aff[c]
                z[c][p] = jnp.maximum((x[c] * a + cc) * m, 0.0)
        zs = [[[_sh(_sh(z[c][p], th, 0), tw, 1)
                for tw in range(2)] for th in range(2)]
              for c in range(M) for p in range(4)]
        for co in range(M):
            acc = None
            for pp in (0, 2):
                for td in range(2):
                    for c in range(M):
                        for th in range(2):
                            for tw in range(2):
                                w = cw_ref[(td * 2 + th) * 2 + tw, c * M + co]
                                t = zs[c * 4 + pp + td][th][tw] * w
                                acc = t if acc is None else acc + t
            acc = acc + _sh(acc, 2, 1)
            acc = acc + _sh(acc, 2, 0)
            y_ref[0, co, f] = jnp.dot(
                jnp.dot(l0_ref[...], acc), r0_ref[...],
                preferred_element_type=jnp.float32) * 0.125
        mp = [(ck[0, d0i + j] > 0).astype(jnp.float32) for j in range(4)]
        mm = jnp.maximum(jnp.maximum(mp[0], mp[1]),
                         jnp.maximum(mp[2], mp[3]))
        mm = jnp.maximum(mm, _sh(mm, 1, 1))
        mm = jnp.maximum(mm, _sh(mm, 2, 1))
        mm = jnp.maximum(mm, _sh(mm, 1, 0))
        mm = jnp.maximum(mm, _sh(mm, 2, 0))
        m_ref[0, f] = jnp.dot(jnp.dot(l0_ref[...] * 1.0, mm), r0_ref[...],
                              preferred_element_type=jnp.float32)


def _mid(dense, cnt, W_sub, bn_w0, bn_b0, W_conv0):
    d4 = dense.reshape(B, S, S, S)
    c4 = cnt.reshape(B, S, S, S)
    nk = S // DBLK
    w27 = W_sub.reshape(27, M)
    blk = (1, DBLK, S, S)
    dm1 = pl.BlockSpec(blk, lambda b, k: (b, jnp.maximum(k - 1, 0), 0, 0))
    dc = pl.BlockSpec(blk, lambda b, k: (b, k, 0, 0))
    dp1 = pl.BlockSpec(blk, lambda b, k: (b, jnp.minimum(k + 1, nk - 1), 0, 0))
    smem = pl.BlockSpec(memory_space=pltpu.SMEM)
    stats = pl.pallas_call(
        _k1_kernel,
        grid=(B, nk),
        in_specs=[dm1, dc, dp1, dc, smem],
        out_specs=pl.BlockSpec((8, 128), lambda b, k: (0, 0)),
        out_shape=jax.ShapeDtypeStruct((8, 128), jnp.float32),
    )(d4, d4, d4, c4, w27)

    sel = (jnp.arange(128)[None, :] == 4 * jnp.arange(32)[:, None]
           ).astype(jnp.float32)
    y32, m32 = pl.pallas_call(
        _k2_kernel,
        grid=(B, nk),
        in_specs=[dm1, dc, dp1, dc,
                  pl.BlockSpec((8, 128), lambda b, k: (0, 0)),
                  smem, smem, smem, smem,
                  pl.BlockSpec((32, 128), lambda b, k: (0, 0)),
                  pl.BlockSpec((128, 32), lambda b, k: (0, 0))],
        out_specs=[pl.BlockSpec((1, M, DBLK // 4, 32, 32),
                                lambda b, k: (b, 0, k, 0, 0)),
                   pl.BlockSpec((1, DBLK // 4, 32, 32),
                                lambda b, k: (b, k, 0, 0))],
        out_shape=[jax.ShapeDtypeStruct((B, M, 32, 32, 32), jnp.float32),
                   jax.ShapeDtypeStruct((B, 32, 32, 32), jnp.float32)],
    )(d4, d4, d4, c4, stats, w27, bn_w0, bn_b0, W_conv0.reshape(8, 16),
      sel, sel.T)
    return y32, m32


def _sh(x, t, axis):
    if t == 0:
        return x
    pad = [(0, 0)] * x.ndim
    pad[axis] = (0, t)
    xp = jnp.pad(x, pad)
    idx = [slice(None)] * x.ndim
    idx[axis] = slice(t, t + x.shape[axis])
    return xp[tuple(idx)]


def _tail_kernel(y_ref, m_ref, bn1w_ref, bn1b_ref, cw1_ref, bn2w_ref,
                 bn2b_ref, cw2_ref, l8_ref, r8_ref, l8m_ref, r8m_ref,
                 l2_ref, r2_ref, out_ref):
    n1 = jnp.maximum(m_ref[0].sum() + m_ref[1].sum(), 1.0)
    y = [[y_ref[b * M + c] for c in range(M)] for b in range(B)]
    z = [[None] * M for _ in range(B)]
    for c in range(M):
        s1 = sum(y[b][c].sum() for b in range(B))
        s2 = sum((y[b][c] * y[b][c]).sum() for b in range(B))
        mean = s1 / n1
        var = s2 / n1 - mean * mean
        a = bn1w_ref[c] * lax.rsqrt(var + EPS)
        cc = bn1b_ref[c] - mean * a
        for b in range(B):
            z[b][c] = jnp.maximum((y[b][c] * a + cc) * m_ref[b], 0.0)

    y8 = [[None] * M for _ in range(B)]
    m8 = [None] * B
    for b in range(B):
        mm = jnp.maximum(m_ref[b], _sh(m_ref[b], 1, 2))
        mm = jnp.maximum(mm, _sh(mm, 1, 1))
        mm = jnp.maximum(mm, _sh(mm, 1, 0))
        m8[b] = jnp.stack([
            jnp.dot(jnp.dot(l8m_ref[...], mm[4 * d]), r8m_ref[...],
                    preferred_element_type=jnp.float32) for d in range(8)])
        for co in range(M):
            acc = jnp.zeros((32, 32, 32), jnp.float32)
            for td in range(2):
                for th in range(2):
                    for tw in range(2):
                        for ci in range(M):
                            w = cw1_ref[td * 4 + th * 2 + tw, ci * M + co]
                            zs = _sh(_sh(_sh(z[b][ci], tw, 2), th, 1), td, 0)
                            acc = acc + zs * w
            planes = []
            for d in range(8):
                p = acc[4 * d] + acc[4 * d + 2]
                planes.append(jnp.dot(jnp.dot(l8_ref[...], p), r8_ref[...],
                                      preferred_element_type=jnp.float32))
            y8[b][co] = jnp.stack(planes) * 0.5

    n2 = jnp.maximum(sum(jnp.sum(m8[b]) for b in range(B)), 1.0)
    z2 = [[None] * M for _ in range(B)]
    for c in range(M):
        s1 = sum(y8[b][c].sum() for b in range(B))
        s2 = sum((y8[b][c] * y8[b][c]).sum() for b in range(B))
        mean = s1 / n2
        var = s2 / n2 - mean * mean
        a = bn2w_ref[c] * lax.rsqrt(var + EPS)
        cc = bn2b_ref[c] - mean * a
        for b in range(B):
            z2[b][c] = jnp.maximum((y8[b][c] * a + cc) * m8[b], 0.0)

    for b in range(B):
        for co in range(M):
            acc = jnp.zeros((8, 8, 8), jnp.float32)
            for td in range(2):
                for th in range(2):
                    for tw in range(2):
                        for ci in range(M):
                            w = cw2_ref[td * 4 + th * 2 + tw, ci * M + co]
                            zs = _sh(_sh(_sh(z2[b][ci], tw, 2), th, 1), td, 0)
                            acc = acc + zs * w
            planes = []
            for d in range(2):
                p = acc[4 * d] + acc[4 * d + 2]
                planes.append(jnp.dot(jnp.dot(l2_ref[...], p), r2_ref[...],
                                      preferred_element_type=jnp.float32))
            out_ref[b * M + co] = jnp.stack(planes) * 0.5


def _tail(y32, m32, bn_w1, bn_b1, W_conv1, bn_w2, bn_b2, W_conv2):
    idx8 = jnp.arange(32)
    l8 = ((idx8[None, :] == 4 * jnp.arange(8)[:, None]) |
          (idx8[None, :] == 4 * jnp.arange(8)[:, None] + 2)).astype(jnp.float32) * 0.5
    r8 = l8.T
    l8m = (idx8[None, :] == 4 * jnp.arange(8)[:, None]).astype(jnp.float32)
    r8m = l8m.T
    idx2 = jnp.arange(8)
    l2 = ((idx2[None, :] == 4 * jnp.arange(2)[:, None]) |
          (idx2[None, :] == 4 * jnp.arange(2)[:, None] + 2)).astype(jnp.float32) * 0.5
    r2 = l2.T
    cw1 = W_conv1.reshape(8, 16)
    cw2 = W_conv2.reshape(8, 16)
    smem = pl.BlockSpec(memory_space=pltpu.SMEM)
    out = pl.pallas_call(
        _tail_kernel,
        out_shape=jax.ShapeDtypeStruct((B * M, 2, 2, 2), jnp.float32),
        in_specs=[pl.BlockSpec(memory_space=pltpu.VMEM)] * 2 +
                 [smem, smem, smem, smem, smem, smem] +
                 [pl.BlockSpec(memory_space=pltpu.VMEM)] * 6,
        out_specs=pl.BlockSpec(memory_space=pltpu.VMEM),
    )(y32.reshape(B * M, 32, 32, 32), m32, bn_w1, bn_b1, cw1,
      bn_w2, bn_b2, cw2, l8, r8, l8m, r8m, l2, r2)
    return out.reshape(B, M * 8)


def kernel(point_cloud, W_sub, bn_w0, bn_b0, W_conv0, bn_w1, bn_b1, W_conv1,
           bn_w2, bn_b2, W_conv2):
    pc = point_cloud
    ix = pc[:, 0].astype(jnp.int32)
    iy = pc[:, 1].astype(jnp.int32)
    iz = pc[:, 2].astype(jnp.int32)
    ib = pc[:, 3].astype(jnp.int32)
    lin = ((ib * S + ix) * S + iy) * S + iz
    npts = lin.shape[0]
    lin_pad = jnp.full((NPTS_PAD,), B * S * S * S, jnp.int32).at[:npts].set(lin)
    feat_pad = jnp.zeros((NPTS_PAD,), jnp.float32).at[:npts].set(pc[:, 4])
    dense, cnt = _sc_voxelize(lin_pad, feat_pad)
    y32, m32 = _mid(dense, cnt, W_sub, bn_w0, bn_b0, W_conv0)
    return _tail(y32, m32, bn_w1, bn_b1, W_conv1, bn_w2, bn_b2, W_conv2)


# speedup vs baseline: 17.4126x; 2.7594x over previous
import functools
import jax
import jax.numpy as jnp
from jax import lax
from jax.experimental import pallas as pl
from jax.experimental.pallas import tpu as pltpu
from jax.experimental.pallas import tpu_sc as plsc

S = 128
B = 2
M = 4
EPS = 1e-4

NC, NS, L = 2, 16, 16
NPTS_PAD = 100352
ROWS_PT = NPTS_PAD // (NC * NS * 128) * NC
QROWS = 49
QUART = S * S * S // 4
TSLICE = QUART // NS


def _sc_scatter_body(idx_hbm, feat_hbm, sum_hbm, cnt_hbm,
                     idxb, featb, relb, onesb, zbuf, spm_sum, spm_cnt, sem):
    cid = lax.axis_index("c")
    tid = lax.axis_index("s")
    pltpu.sync_copy(idx_hbm.at[tid], idxb)
    pltpu.sync_copy(feat_hbm.at[tid], featb)
    for i in range(128 // L):
        onesb[pl.ds(i * L, L)] = jnp.ones((L,), jnp.float32)
    def _z(i, carry):
        zbuf[pl.ds(i * L, L)] = jnp.zeros((L,), jnp.float32)
        return carry
    lax.fori_loop(0, TSLICE // L, _z, 0)

    for q in range(4):
        base = cid * (S * S * S) + q * QUART
        pltpu.sync_copy(zbuf, spm_sum.at[pl.ds(tid * TSLICE, TSLICE)])
        pltpu.sync_copy(zbuf, spm_cnt.at[pl.ds(tid * TSLICE, TSLICE)])
        plsc.subcore_barrier()

        def _rel(j, carry):
            for i in range(128 // L):
                v = idxb[j, pl.ds(i * L, L)] - base
                ok = (v >= 0) & (v < QUART)
                relb[j, pl.ds(i * L, L)] = jnp.where(ok, v, QUART)
            return carry
        lax.fori_loop(0, QROWS, _rel, 0, unroll=2)

        for j0 in range(0, QROWS, 8):
            copies = []
            for j in range(j0, min(j0 + 8, QROWS)):
                copies.append(pltpu.async_copy(
                    featb.at[j], spm_sum.at[relb.at[j]], sem, add=True))
                copies.append(pltpu.async_copy(
                    onesb, spm_cnt.at[relb.at[j]], sem, add=True))
            for cp in copies:
                cp.wait()
        plsc.subcore_barrier()

        pltpu.sync_copy(spm_sum.at[pl.ds(tid * TSLICE, TSLICE)],
                        sum_hbm.at[pl.ds(base + tid * TSLICE, TSLICE)])
        pltpu.sync_copy(spm_cnt.at[pl.ds(tid * TSLICE, TSLICE)],
                        cnt_hbm.at[pl.ds(base + tid * TSLICE, TSLICE)])
        plsc.subcore_barrier()


def _sc_voxelize(lin_idx, feat):
    idx2d = lin_idx.reshape(NS, QROWS, 128)
    feat2d = feat.reshape(NS, QROWS, 128)
    mesh = plsc.VectorSubcoreMesh(core_axis_name="c", subcore_axis_name="s",
                                  num_cores=NC, num_subcores=NS)
    f = pl.kernel(
        _sc_scatter_body,
        mesh=mesh,
        out_type=[jax.ShapeDtypeStruct((B * S * S * S,), jnp.float32),
                  jax.ShapeDtypeStruct((B * S * S * S,), jnp.float32)],
        scratch_types=[
            pltpu.VMEM((QROWS, 128), jnp.int32),
            pltpu.VMEM((QROWS, 128), jnp.float32),
            pltpu.VMEM((QROWS, 128), jnp.int32),
            pltpu.VMEM((128,), jnp.float32),
            pltpu.VMEM((TSLICE,), jnp.float32),
            pltpu.VMEM_SHARED((QUART + 128,), jnp.float32),
            pltpu.VMEM_SHARED((QUART + 128,), jnp.float32),
            pltpu.SemaphoreType.DMA,
        ],
    )
    return f(idx2d, feat2d)


DBLK = 16


def _make_volp(dm1_ref, d0_ref, dp1_ref, k, nk):
    zero = jnp.zeros((1, S, S), jnp.float32)
    prev = jnp.where(k > 0, dm1_ref[0, DBLK - 1:DBLK], zero)
    nxt = jnp.where(k < nk - 1, dp1_ref[0, 0:1], zero)
    vol = jnp.concatenate([prev, d0_ref[0], nxt], axis=0)
    return jnp.pad(vol, ((0, 0), (1, 1), (0, 0)))


def _conv_plane(volp, d, mb_ref):
    acc = None
    for dd in range(3):
        for dh in range(3):
            a = volp[d + dd, dh:dh + S, :]
            t = jnp.dot(a, mb_ref[dd * 3 + dh],
                        preferred_element_type=jnp.float32)
            acc = t if acc is None else acc + t
    return [acc[:, c * S:(c + 1) * S] for c in range(M)]


def _band_mats(W_sub):
    cols = []
    for c in range(M):
        band = sum(W_sub[:, :, dw, 0, c][:, :, None, None]
                   * jnp.eye(S, k=1 - dw)[None, None] for dw in range(3))
        cols.append(band)
    full = jnp.concatenate(cols, axis=-1)
    return full.reshape(9, S, M * S)


def _stats_update(out_ref, first, s1, s2, n):
    row = lax.broadcasted_iota(jnp.int32, (8, 128), 0)
    lane = lax.broadcasted_iota(jnp.int32, (8, 128), 1)
    upd = jnp.where((row == 2) & (lane == 0), n, 0.0)
    for c in range(M):
        upd = jnp.where((row == 0) & (lane == c), s1[c], upd)
        upd = jnp.where((row == 1) & (lane == c), s2[c], upd)
    @pl.when(first)
    def _():
        out_ref[...] = jnp.zeros((8, 128), jnp.float32)
    out_ref[...] += upd


def _k1_kernel(dm1, d0, dp1, ck, mb_ref, out_ref):
    b, k = pl.program_id(0), pl.program_id(1)
    volp = _make_volp(dm1, d0, dp1, k, S // DBLK)
    s1 = [jnp.float32(0.0)] * M
    s2 = [jnp.float32(0.0)] * M
    n = jnp.float32(0.0)
    for d in range(DBLK):
        x = _conv_plane(volp, d, mb_ref)
        m = (ck[0, d] > 0).astype(jnp.float32)
        n = n + m.sum()
        for c in range(M):
            xm = x[c] * m
            s1[c] = s1[c] + xm.sum()
            s2[c] = s2[c] + (xm * xm).sum()
    _stats_update(out_ref, (b == 0) & (k == 0), s1, s2, n)


def _k2_kernel(dm1, d0, dp1, ck, stats_ref, mb_ref, bw_ref, bb_ref, cw_ref,
               l0_ref, r0_ref, y_ref, m_ref):
    b, k = pl.program_id(0), pl.program_id(1)
    volp = _make_volp(dm1, d0, dp1, k, S // DBLK)
    n = jnp.maximum(stats_ref[2, 0], 1.0)
    aff = []
    for c in range(M):
        mean = stats_ref[0, c] / n
        var = stats_ref[1, c] / n - mean * mean
        a = bw_ref[c] * lax.rsqrt(var + EPS)
        aff.append((a, bb_ref[c] - mean * a))
    for f in range(DBLK // 4):
        d0i = 4 * f
        z = [[None] * 4 for _ in range(M)]
        for p in range(4):
            x = _conv_plane(volp, d0i + p, mb_ref)
            m = (ck[0, d0i + p] > 0).astype(jnp.float32)
            for c in range(M):
                a, cc = aff[c]
                z[c][p] = jnp.maximum((x[c] * a + cc) * m, 0.0)
        zps = [[[[None] * 2 for _ in range(2)] for _ in range(2)]
               for _ in range(M)]
        for c in range(M):
            for td in range(2):
                zp = z[c][td] + z[c][2 + td]
                for th in range(2):
                    for tw in range(2):
                        zps[c][td][th][tw] = _sh(_sh(zp, th, 0), tw, 1)
        for co in range(M):
            acc = None
            for c in range(M):
                for td in range(2):
                    for th in range(2):
                        for tw in range(2):
                            w = cw_ref[(td * 2 + th) * 2 + tw, c * M + co]
                            t = zps[c][td][th][tw] * w
                            acc = t if acc is None else acc + t
            acc = acc + _sh(acc, 2, 1)
            acc = acc + _sh(acc, 2, 0)
            y_ref[0, co, f] = jnp.dot(
                jnp.dot(l0_ref[...], acc), r0_ref[...],
                preferred_element_type=jnp.float32) * 0.125
        mp = [(ck[0, d0i + j] > 0).astype(jnp.float32) for j in range(4)]
        mm = jnp.maximum(jnp.maximum(mp[0], mp[1]),
                         jnp.maximum(mp[2], mp[3]))
        mm = jnp.maximum(mm, _sh(mm, 1, 1))
        mm = jnp.maximum(mm, _sh(mm, 2, 1))
        mm = jnp.maximum(mm, _sh(mm, 1, 0))
        mm = jnp.maximum(mm, _sh(mm, 2, 0))
        m_ref[0, f] = jnp.dot(jnp.dot(l0_ref[...] * 1.0, mm), r0_ref[...],
                              preferred_element_type=jnp.float32)


def _mid(dense, cnt, W_sub, bn_w0, bn_b0, W_conv0):
    d4 = dense.reshape(B, S, S, S)
    c4 = cnt.reshape(B, S, S, S)
    nk = S // DBLK
    mb = _band_mats(W_sub)
    mbspec = pl.BlockSpec((9, S, M * S), lambda b, k: (0, 0, 0))
    blk = (1, DBLK, S, S)
    dm1 = pl.BlockSpec(blk, lambda b, k: (b, jnp.maximum(k - 1, 0), 0, 0))
    dc = pl.BlockSpec(blk, lambda b, k: (b, k, 0, 0))
    dp1 = pl.BlockSpec(blk, lambda b, k: (b, jnp.minimum(k + 1, nk - 1), 0, 0))
    smem = pl.BlockSpec(memory_space=pltpu.SMEM)
    stats = pl.pallas_call(
        _k1_kernel,
        grid=(B, nk),
        in_specs=[dm1, dc, dp1, dc, mbspec],
        out_specs=pl.BlockSpec((8, 128), lambda b, k: (0, 0)),
        out_shape=jax.ShapeDtypeStruct((8, 128), jnp.float32),
    )(d4, d4, d4, c4, mb)

    sel = (jnp.arange(128)[None, :] == 4 * jnp.arange(32)[:, None]
           ).astype(jnp.float32)
    y32, m32 = pl.pallas_call(
        _k2_kernel,
        grid=(B, nk),
        in_specs=[dm1, dc, dp1, dc,
                  pl.BlockSpec((8, 128), lambda b, k: (0, 0)),
                  mbspec, smem, smem, smem,
                  pl.BlockSpec((32, 128), lambda b, k: (0, 0)),
                  pl.BlockSpec((128, 32), lambda b, k: (0, 0))],
        out_specs=[pl.BlockSpec((1, M, DBLK // 4, 32, 32),
                                lambda b, k: (b, 0, k, 0, 0)),
                   pl.BlockSpec((1, DBLK // 4, 32, 32),
                                lambda b, k: (b, k, 0, 0))],
        out_shape=[jax.ShapeDtypeStruct((B, M, 32, 32, 32), jnp.float32),
                   jax.ShapeDtypeStruct((B, 32, 32, 32), jnp.float32)],
    )(d4, d4, d4, c4, stats, mb, bn_w0, bn_b0, W_conv0.reshape(8, 16),
      sel, sel.T)
    return y32, m32


def _sh(x, t, axis):
    if t == 0:
        return x
    pad = [(0, 0)] * x.ndim
    pad[axis] = (0, t)
    xp = jnp.pad(x, pad)
    idx = [slice(None)] * x.ndim
    idx[axis] = slice(t, t + x.shape[axis])
    return xp[tuple(idx)]


def _tail_kernel(y_ref, m_ref, bn1w_ref, bn1b_ref, cw1_ref, bn2w_ref,
                 bn2b_ref, cw2_ref, l8_ref, r8_ref, l8m_ref, r8m_ref,
                 l2_ref, r2_ref, out_ref):
    n1 = jnp.maximum(m_ref[0].sum() + m_ref[1].sum(), 1.0)
    y = [[y_ref[b * M + c] for c in range(M)] for b in range(B)]
    z = [[None] * M for _ in range(B)]
    for c in range(M):
        s1 = sum(y[b][c].sum() for b in range(B))
        s2 = sum((y[b][c] * y[b][c]).sum() for b in range(B))
        mean = s1 / n1
        var = s2 / n1 - mean * mean
        a = bn1w_ref[c] * lax.rsqrt(var + EPS)
        cc = bn1b_ref[c] - mean * a
        for b in range(B):
            z[b][c] = jnp.maximum((y[b][c] * a + cc) * m_ref[b], 0.0)

    y8 = [[None] * M for _ in range(B)]
    m8 = [None] * B
    for b in range(B):
        mm = jnp.maximum(m_ref[b], _sh(m_ref[b], 1, 2))
        mm = jnp.maximum(mm, _sh(mm, 1, 1))
        mm = jnp.maximum(mm, _sh(mm, 1, 0))
        m8[b] = jnp.stack([
            jnp.dot(jnp.dot(l8m_ref[...], mm[4 * d]), r8m_ref[...],
                    preferred_element_type=jnp.float32) for d in range(8)])
        for co in range(M):
            acc = jnp.zeros((32, 32, 32), jnp.float32)
            for td in range(2):
                for th in range(2):
                    for tw in range(2):
                        for ci in range(M):
                            w = cw1_ref[td * 4 + th * 2 + tw, ci * M + co]
                            zs = _sh(_sh(_sh(z[b][ci], tw, 2), th, 1), td, 0)
                            acc = acc + zs * w
            planes = []
            for d in range(8):
                p = acc[4 * d] + acc[4 * d + 2]
                planes.append(jnp.dot(jnp.dot(l8_ref[...], p), r8_ref[...],
                                      preferred_element_type=jnp.float32))
            y8[b][co] = jnp.stack(planes) * 0.5

    n2 = jnp.maximum(sum(jnp.sum(m8[b]) for b in range(B)), 1.0)
    z2 = [[None] * M for _ in range(B)]
    for c in range(M):
        s1 = sum(y8[b][c].sum() for b in range(B))
        s2 = sum((y8[b][c] * y8[b][c]).sum() for b in range(B))
        mean = s1 / n2
        var = s2 / n2 - mean * mean
        a = bn2w_ref[c] * lax.rsqrt(var + EPS)
        cc = bn2b_ref[c] - mean * a
        for b in range(B):
            z2[b][c] = jnp.maximum((y8[b][c] * a + cc) * m8[b], 0.0)

    for b in range(B):
        for co in range(M):
            acc = jnp.zeros((8, 8, 8), jnp.float32)
            for td in range(2):
                for th in range(2):
                    for tw in range(2):
                        for ci in range(M):
                            w = cw2_ref[td * 4 + th * 2 + tw, ci * M + co]
                            zs = _sh(_sh(_sh(z2[b][ci], tw, 2), th, 1), td, 0)
                            acc = acc + zs * w
            planes = []
            for d in range(2):
                p = acc[4 * d] + acc[4 * d + 2]
                planes.append(jnp.dot(jnp.dot(l2_ref[...], p), r2_ref[...],
                                      preferred_element_type=jnp.float32))
            out_ref[b * M + co] = jnp.stack(planes) * 0.5


def _tail(y32, m32, bn_w1, bn_b1, W_conv1, bn_w2, bn_b2, W_conv2):
    idx8 = jnp.arange(32)
    l8 = ((idx8[None, :] == 4 * jnp.arange(8)[:, None]) |
          (idx8[None, :] == 4 * jnp.arange(8)[:, None] + 2)).astype(jnp.float32) * 0.5
    r8 = l8.T
    l8m = (idx8[None, :] == 4 * jnp.arange(8)[:, None]).astype(jnp.float32)
    r8m = l8m.T
    idx2 = jnp.arange(8)
    l2 = ((idx2[None, :] == 4 * jnp.arange(2)[:, None]) |
          (idx2[None, :] == 4 * jnp.arange(2)[:, None] + 2)).astype(jnp.float32) * 0.5
    r2 = l2.T
    cw1 = W_conv1.reshape(8, 16)
    cw2 = W_conv2.reshape(8, 16)
    smem = pl.BlockSpec(memory_space=pltpu.SMEM)
    out = pl.pallas_call(
        _tail_kernel,
        out_shape=jax.ShapeDtypeStruct((B * M, 2, 2, 2), jnp.float32),
        in_specs=[pl.BlockSpec(memory_space=pltpu.VMEM)] * 2 +
                 [smem, smem, smem, smem, smem, smem] +
                 [pl.BlockSpec(memory_space=pltpu.VMEM)] * 6,
        out_specs=pl.BlockSpec(memory_space=pltpu.VMEM),
    )(y32.reshape(B * M, 32, 32, 32), m32, bn_w1, bn_b1, cw1,
      bn_w2, bn_b2, cw2, l8, r8, l8m, r8m, l2, r2)
    return out.reshape(B, M * 8)


def kernel(point_cloud, W_sub, bn_w0, bn_b0, W_conv0, bn_w1, bn_b1, W_conv1,
           bn_w2, bn_b2, W_conv2):
    pc = point_cloud
    ix = pc[:, 0].astype(jnp.int32)
    iy = pc[:, 1].astype(jnp.int32)
    iz = pc[:, 2].astype(jnp.int32)
    ib = pc[:, 3].astype(jnp.int32)
    lin = ((ib * S + ix) * S + iy) * S + iz
    npts = lin.shape[0]
    lin_pad = jnp.full((NPTS_PAD,), B * S * S * S, jnp.int32).at[:npts].set(lin)
    feat_pad = jnp.zeros((NPTS_PAD,), jnp.float32).at[:npts].set(pc[:, 4])
    dense, cnt = _sc_voxelize(lin_pad, feat_pad)
    y32, m32 = _mid(dense, cnt, W_sub, bn_w0, bn_b0, W_conv0)
    return _tail(y32, m32, bn_w1, bn_b1, W_conv1, bn_w2, bn_b2, W_conv2)


# speedup vs baseline: 42.5702x; 2.4448x over previous
import functools
import jax
import jax.numpy as jnp
from jax import lax
from jax.experimental import pallas as pl
from jax.experimental.pallas import tpu as pltpu
from jax.experimental.pallas import tpu_sc as plsc

S = 128
B = 2
M = 4
EPS = 1e-4

NC, NS, L = 2, 16, 16
NPTS_PAD = 100352
ROWS_PT = NPTS_PAD // (NC * NS * 128) * NC
QROWS = 49
QUART = S * S * S // 4
TSLICE = QUART // NS


def _sc_scatter_body(idx_hbm, feat_hbm, sum_hbm, cnt_hbm,
                     idxb, featb, relb, onesb, zbuf, spm_sum, spm_cnt, sem):
    cid = lax.axis_index("c")
    tid = lax.axis_index("s")
    pltpu.sync_copy(idx_hbm.at[tid], idxb)
    pltpu.sync_copy(feat_hbm.at[tid], featb)
    for i in range(128 // L):
        onesb[pl.ds(i * L, L)] = jnp.ones((L,), jnp.float32)
    def _z(i, carry):
        zbuf[pl.ds(i * L, L)] = jnp.zeros((L,), jnp.float32)
        return carry
    lax.fori_loop(0, TSLICE // L, _z, 0)

    dump = QUART + tid * L + lax.iota(jnp.int32, L)

    for q in range(4):
        base = cid * (S * S * S) + q * QUART
        pltpu.sync_copy(zbuf, spm_sum.at[pl.ds(tid * TSLICE, TSLICE)])
        pltpu.sync_copy(zbuf, spm_cnt.at[pl.ds(tid * TSLICE, TSLICE)])
        plsc.subcore_barrier()

        def _rel(j, carry):
            for i in range(128 // L):
                v = idxb[j, pl.ds(i * L, L)] - base
                ok = (v >= 0) & (v < QUART)
                relb[j, pl.ds(i * L, L)] = jnp.where(ok, v, dump)
            return carry
        lax.fori_loop(0, QROWS, _rel, 0, unroll=2)

        for j0 in range(0, QROWS, 8):
            copies = []
            for j in range(j0, min(j0 + 8, QROWS)):
                copies.append(pltpu.async_copy(
                    featb.at[j], spm_sum.at[relb.at[j]], sem, add=True))
                copies.append(pltpu.async_copy(
                    onesb, spm_cnt.at[relb.at[j]], sem, add=True))
            for cp in copies:
                cp.wait()
        plsc.subcore_barrier()

        pltpu.sync_copy(spm_sum.at[pl.ds(tid * TSLICE, TSLICE)],
                        sum_hbm.at[pl.ds(base + tid * TSLICE, TSLICE)])
        pltpu.sync_copy(spm_cnt.at[pl.ds(tid * TSLICE, TSLICE)],
                        cnt_hbm.at[pl.ds(base + tid * TSLICE, TSLICE)])
        plsc.subcore_barrier()


def _sc_voxelize(lin_idx, feat):
    idx2d = lin_idx.reshape(NS, QROWS, 128)
    feat2d = feat.reshape(NS, QROWS, 128)
    mesh = plsc.VectorSubcoreMesh(core_axis_name="c", subcore_axis_name="s",
                                  num_cores=NC, num_subcores=NS)
    f = pl.kernel(
        _sc_scatter_body,
        mesh=mesh,
        out_type=[jax.ShapeDtypeStruct((B * S * S * S,), jnp.float32),
                  jax.ShapeDtypeStruct((B * S * S * S,), jnp.float32)],
        scratch_types=[
            pltpu.VMEM((QROWS, 128), jnp.int32),
            pltpu.VMEM((QROWS, 128), jnp.float32),
            pltpu.VMEM((QROWS, 128), jnp.int32),
            pltpu.VMEM((128,), jnp.float32),
            pltpu.VMEM((TSLICE,), jnp.float32),
            pltpu.VMEM_SHARED((QUART + NS * L,), jnp.float32),
            pltpu.VMEM_SHARED((QUART + NS * L,), jnp.float32),
            pltpu.SemaphoreType.DMA,
        ],
    )
    return f(idx2d, feat2d)


DBLK = 16


def _make_volp(dm1_ref, d0_ref, dp1_ref, k, nk):
    zero = jnp.zeros((1, S, S), jnp.float32)
    prev = jnp.where(k > 0, dm1_ref[0, DBLK - 1:DBLK], zero)
    nxt = jnp.where(k < nk - 1, dp1_ref[0, 0:1], zero)
    vol = jnp.concatenate([prev, d0_ref[0], nxt], axis=0)
    return jnp.pad(vol, ((0, 0), (1, 1), (0, 0)))


def _conv_plane(volp, d, mb_ref):
    acc = None
    for dd in range(3):
        for dh in range(3):
            a = volp[d + dd, dh:dh + S, :]
            t = jnp.dot(a, mb_ref[dd * 3 + dh],
                        preferred_element_type=jnp.float32)
            acc = t if acc is None else acc + t
    return [acc[:, c * S:(c + 1) * S] for c in range(M)]


def _band_mats(W_sub):
    cols = []
    for c in range(M):
        band = sum(W_sub[:, :, dw, 0, c][:, :, None, None]
                   * jnp.eye(S, k=1 - dw)[None, None] for dw in range(3))
        cols.append(band)
    full = jnp.concatenate(cols, axis=-1)
    return full.reshape(9, S, M * S)


def _stats_update(out_ref, first, s1, s2, n):
    row = lax.broadcasted_iota(jnp.int32, (8, 128), 0)
    lane = lax.broadcasted_iota(jnp.int32, (8, 128), 1)
    upd = jnp.where((row == 2) & (lane == 0), n, 0.0)
    for c in range(M):
        upd = jnp.where((row == 0) & (lane == c), s1[c], upd)
        upd = jnp.where((row == 1) & (lane == c), s2[c], upd)
    @pl.when(first)
    def _():
        out_ref[...] = jnp.zeros((8, 128), jnp.float32)
    out_ref[...] += upd


def _k1_kernel(dm1, d0, dp1, ck, mb_ref, out_ref):
    b, k = pl.program_id(0), pl.program_id(1)
    volp = _make_volp(dm1, d0, dp1, k, S // DBLK)
    s1 = [jnp.float32(0.0)] * M
    s2 = [jnp.float32(0.0)] * M
    n = jnp.float32(0.0)
    for d in range(DBLK):
        x = _conv_plane(volp, d, mb_ref)
        m = (ck[0, d] > 0).astype(jnp.float32)
        n = n + m.sum()
        for c in range(M):
            xm = x[c] * m
            s1[c] = s1[c] + xm.sum()
            s2[c] = s2[c] + (xm * xm).sum()
    _stats_update(out_ref, (b == 0) & (k == 0), s1, s2, n)


def _k2_kernel(dm1, d0, dp1, ck, stats_ref, mb_ref, bw_ref, bb_ref, cw_ref,
               l0_ref, r0_ref, y_ref, m_ref):
    b, k = pl.program_id(0), pl.program_id(1)
    volp = _make_volp(dm1, d0, dp1, k, S // DBLK)
    n = jnp.maximum(stats_ref[2, 0], 1.0)
    aff = []
    for c in range(M):
        mean = stats_ref[0, c] / n
        var = stats_ref[1, c] / n - mean * mean
        a = bw_ref[c] * lax.rsqrt(var + EPS)
        aff.append((a, bb_ref[c] - mean * a))
    for f in range(DBLK // 4):
        d0i = 4 * f
        z = [[None] * 4 for _ in range(M)]
        for p in range(4):
            x = _conv_plane(volp, d0i + p, mb_ref)
            m = (ck[0, d0i + p] > 0).astype(jnp.float32)
            for c in range(M):
                a, cc = aff[c]
                z[c][p] = jnp.maximum((x[c] * a + cc) * m, 0.0)
        zps = [[[[None] * 2 for _ in range(2)] for _ in range(2)]
               for _ in range(M)]
        for c in range(M):
            for td in range(2):
                zp = z[c][td] + z[c][2 + td]
                for th in range(2):
                    for tw in range(2):
                        zps[c][td][th][tw] = _sh(_sh(zp, th, 0), tw, 1)
        for co in range(M):
            acc = None
            for c in range(M):
                for td in range(2):
                    for th in range(2):
                        for tw in range(2):
                            w = cw_ref[(td * 2 + th) * 2 + tw, c * M + co]
                            t = zps[c][td][th][tw] * w
                            acc = t if acc is None else acc + t
            acc = acc + _sh(acc, 2, 1)
            acc = acc + _sh(acc, 2, 0)
            y_ref[0, co, f] = jnp.dot(
                jnp.dot(l0_ref[...], acc), r0_ref[...],
                preferred_element_type=jnp.float32) * 0.125
        mp = [(ck[0, d0i + j] > 0).astype(jnp.float32) for j in range(4)]
        mm = jnp.maximum(jnp.maximum(mp[0], mp[1]),
                         jnp.maximum(mp[2], mp[3]))
        mm = jnp.maximum(mm, _sh(mm, 1, 1))
        mm = jnp.maximum(mm, _sh(mm, 2, 1))
        mm = jnp.maximum(mm, _sh(mm, 1, 0))
        mm = jnp.maximum(mm, _sh(mm, 2, 0))
        m_ref[0, f] = jnp.dot(jnp.dot(l0_ref[...] * 1.0, mm), r0_ref[...],
                              preferred_element_type=jnp.float32)


def _mid(dense, cnt, W_sub, bn_w0, bn_b0, W_conv0):
    d4 = dense.reshape(B, S, S, S)
    c4 = cnt.reshape(B, S, S, S)
    nk = S // DBLK
    mb = _band_mats(W_sub)
    mbspec = pl.BlockSpec((9, S, M * S), lambda b, k: (0, 0, 0))
    blk = (1, DBLK, S, S)
    dm1 = pl.BlockSpec(blk, lambda b, k: (b, jnp.maximum(k - 1, 0), 0, 0))
    dc = pl.BlockSpec(blk, lambda b, k: (b, k, 0, 0))
    dp1 = pl.BlockSpec(blk, lambda b, k: (b, jnp.minimum(k + 1, nk - 1), 0, 0))
    smem = pl.BlockSpec(memory_space=pltpu.SMEM)
    stats = pl.pallas_call(
        _k1_kernel,
        grid=(B, nk),
        in_specs=[dm1, dc, dp1, dc, mbspec],
        out_specs=pl.BlockSpec((8, 128), lambda b, k: (0, 0)),
        out_shape=jax.ShapeDtypeStruct((8, 128), jnp.float32),
    )(d4, d4, d4, c4, mb)

    sel = (jnp.arange(128)[None, :] == 4 * jnp.arange(32)[:, None]
           ).astype(jnp.float32)
    y32, m32 = pl.pallas_call(
        _k2_kernel,
        grid=(B, nk),
        in_specs=[dm1, dc, dp1, dc,
                  pl.BlockSpec((8, 128), lambda b, k: (0, 0)),
                  mbspec, smem, smem, smem,
                  pl.BlockSpec((32, 128), lambda b, k: (0, 0)),
                  pl.BlockSpec((128, 32), lambda b, k: (0, 0))],
        out_specs=[pl.BlockSpec((1, M, DBLK // 4, 32, 32),
                                lambda b, k: (b, 0, k, 0, 0)),
                   pl.BlockSpec((1, DBLK // 4, 32, 32),
                                lambda b, k: (b, k, 0, 0))],
        out_shape=[jax.ShapeDtypeStruct((B, M, 32, 32, 32), jnp.float32),
                   jax.ShapeDtypeStruct((B, 32, 32, 32), jnp.float32)],
    )(d4, d4, d4, c4, stats, mb, bn_w0, bn_b0, W_conv0.reshape(8, 16),
      sel, sel.T)
    return y32, m32


def _sh(x, t, axis):
    if t == 0:
        return x
    pad = [(0, 0)] * x.ndim
    pad[axis] = (0, t)
    xp = jnp.pad(x, pad)
    idx = [slice(None)] * x.ndim
    idx[axis] = slice(t, t + x.shape[axis])
    return xp[tuple(idx)]


def _tail_kernel(y_ref, m_ref, bn1w_ref, bn1b_ref, cw1_ref, bn2w_ref,
                 bn2b_ref, cw2_ref, l8_ref, r8_ref, l8m_ref, r8m_ref,
                 l2_ref, r2_ref, out_ref):
    n1 = jnp.maximum(m_ref[0].sum() + m_ref[1].sum(), 1.0)
    y = [[y_ref[b * M + c] for c in range(M)] for b in range(B)]
    z = [[None] * M for _ in range(B)]
    for c in range(M):
        s1 = sum(y[b][c].sum() for b in range(B))
        s2 = sum((y[b][c] * y[b][c]).sum() for b in range(B))
        mean = s1 / n1
        var = s2 / n1 - mean * mean
        a = bn1w_ref[c] * lax.rsqrt(var + EPS)
        cc = bn1b_ref[c] - mean * a
        for b in range(B):
            z[b][c] = jnp.maximum((y[b][c] * a + cc) * m_ref[b], 0.0)

    y8 = [[None] * M for _ in range(B)]
    m8 = [None] * B
    for b in range(B):
        mm = jnp.maximum(m_ref[b], _sh(m_ref[b], 1, 2))
        mm = jnp.maximum(mm, _sh(mm, 1, 1))
        mm = jnp.maximum(mm, _sh(mm, 1, 0))
        m8[b] = jnp.stack([
            jnp.dot(jnp.dot(l8m_ref[...], mm[4 * d]), r8m_ref[...],
                    preferred_element_type=jnp.float32) for d in range(8)])
        for co in range(M):
            acc = jnp.zeros((32, 32, 32), jnp.float32)
            for td in range(2):
                for th in range(2):
                    for tw in range(2):
                        for ci in range(M):
                            w = cw1_ref[td * 4 + th * 2 + tw, ci * M + co]
                            zs = _sh(_sh(_sh(z[b][ci], tw, 2), th, 1), td, 0)
                            acc = acc + zs * w
            planes = []
            for d in range(8):
                p = acc[4 * d] + acc[4 * d + 2]
                planes.append(jnp.dot(jnp.dot(l8_ref[...], p), r8_ref[...],
                                      preferred_element_type=jnp.float32))
            y8[b][co] = jnp.stack(planes) * 0.5

    n2 = jnp.maximum(sum(jnp.sum(m8[b]) for b in range(B)), 1.0)
    z2 = [[None] * M for _ in range(B)]
    for c in range(M):
        s1 = sum(y8[b][c].sum() for b in range(B))
        s2 = sum((y8[b][c] * y8[b][c]).sum() for b in range(B))
        mean = s1 / n2
        var = s2 / n2 - mean * mean
        a = bn2w_ref[c] * lax.rsqrt(var + EPS)
        cc = bn2b_ref[c] - mean * a
        for b in range(B):
            z2[b][c] = jnp.maximum((y8[b][c] * a + cc) * m8[b], 0.0)

    for b in range(B):
        for co in range(M):
            acc = jnp.zeros((8, 8, 8), jnp.float32)
            for td in range(2):
                for th in range(2):
                    for tw in range(2):
                        for ci in range(M):
                            w = cw2_ref[td * 4 + th * 2 + tw, ci * M + co]
                            zs = _sh(_sh(_sh(z2[b][ci], tw, 2), th, 1), td, 0)
                            acc = acc + zs * w
            planes = []
            for d in range(2):
                p = acc[4 * d] + acc[4 * d + 2]
                planes.append(jnp.dot(jnp.dot(l2_ref[...], p), r2_ref[...],
                                      preferred_element_type=jnp.float32))
            out_ref[b * M + co] = jnp.stack(planes) * 0.5


def _tail(y32, m32, bn_w1, bn_b1, W_conv1, bn_w2, bn_b2, W_conv2):
    idx8 = jnp.arange(32)
    l8 = ((idx8[None, :] == 4 * jnp.arange(8)[:, None]) |
          (idx8[None, :] == 4 * jnp.arange(8)[:, None] + 2)).astype(jnp.float32) * 0.5
    r8 = l8.T
    l8m = (idx8[None, :] == 4 * jnp.arange(8)[:, None]).astype(jnp.float32)
    r8m = l8m.T
    idx2 = jnp.arange(8)
    l2 = ((idx2[None, :] == 4 * jnp.arange(2)[:, None]) |
          (idx2[None, :] == 4 * jnp.arange(2)[:, None] + 2)).astype(jnp.float32) * 0.5
    r2 = l2.T
    cw1 = W_conv1.reshape(8, 16)
    cw2 = W_conv2.reshape(8, 16)
    smem = pl.BlockSpec(memory_space=pltpu.SMEM)
    out = pl.pallas_call(
        _tail_kernel,
        out_shape=jax.ShapeDtypeStruct((B * M, 2, 2, 2), jnp.float32),
        in_specs=[pl.BlockSpec(memory_space=pltpu.VMEM)] * 2 +
                 [smem, smem, smem, smem, smem, smem] +
                 [pl.BlockSpec(memory_space=pltpu.VMEM)] * 6,
        out_specs=pl.BlockSpec(memory_space=pltpu.VMEM),
    )(y32.reshape(B * M, 32, 32, 32), m32, bn_w1, bn_b1, cw1,
      bn_w2, bn_b2, cw2, l8, r8, l8m, r8m, l2, r2)
    return out.reshape(B, M * 8)


def kernel(point_cloud, W_sub, bn_w0, bn_b0, W_conv0, bn_w1, bn_b1, W_conv1,
           bn_w2, bn_b2, W_conv2):
    pc = point_cloud
    ix = pc[:, 0].astype(jnp.int32)
    iy = pc[:, 1].astype(jnp.int32)
    iz = pc[:, 2].astype(jnp.int32)
    ib = pc[:, 3].astype(jnp.int32)
    lin = ((ib * S + ix) * S + iy) * S + iz
    npts = lin.shape[0]
    lin_pad = jnp.full((NPTS_PAD,), B * S * S * S, jnp.int32).at[:npts].set(lin)
    feat_pad = jnp.zeros((NPTS_PAD,), jnp.float32).at[:npts].set(pc[:, 4])
    dense, cnt = _sc_voxelize(lin_pad, feat_pad)
    y32, m32 = _mid(dense, cnt, W_sub, bn_w0, bn_b0, W_conv0)
    return _tail(y32, m32, bn_w1, bn_b1, W_conv1, bn_w2, bn_b2, W_conv2)
